# baseline (speedup 1.0000x reference)
import functools

import jax
import jax.numpy as jnp
from jax import lax
from jax.experimental import pallas as pl
from jax.experimental.pallas import tpu as pltpu
from jax.experimental.pallas import tpu_sc as plsc

B = 16384
D = 32
RPB = 128 // D * 1
NC = 2
NS = 16
NW = NC * NS
BPW = B // NW
NCHUNK = 2
CH = BPW // NCHUNK

_mesh = plsc.VectorSubcoreMesh(core_axis_name="c", subcore_axis_name="s")


@functools.partial(
    pl.kernel,
    mesh=_mesh,
    out_type=jax.ShapeDtypeStruct((B,), jnp.float32),
    scratch_types=[
        pltpu.VMEM((BPW,), jnp.int32),
        pltpu.VMEM((BPW,), jnp.int32),
        pltpu.VMEM((BPW,), jnp.int32),
        pltpu.VMEM((BPW,), jnp.int32),
        pltpu.VMEM((CH, 128), jnp.float32),
        pltpu.VMEM((CH, 128), jnp.float32),
        pltpu.VMEM((BPW,), jnp.float32),
        pltpu.SemaphoreType.DMA,
        pltpu.SemaphoreType.DMA,
    ],
    compiler_params=pltpu.CompilerParams(needs_layout_passes=False),
)
def _sc_dot(uid_hbm, iid_hbm, utab_hbm, itab_hbm, out_hbm,
            uidx_v, iidx_v, ublk_v, iblk_v, ubuf, ibuf, out_v,
            sem_idx, sem):
    wid = lax.axis_index("s") * NC + lax.axis_index("c")
    base = wid * BPW
    pltpu.async_copy(uid_hbm.at[pl.ds(base, BPW)], uidx_v, sem_idx).wait()
    pltpu.async_copy(iid_hbm.at[pl.ds(base, BPW)], iidx_v, sem_idx).wait()

    @pl.loop(0, BPW // 16)
    def _(g):
        sl = pl.ds(g * 16, 16)
        ublk_v[sl] = uidx_v[sl] >> 2
        iblk_v[sl] = iidx_v[sl] >> 2

    lane = lax.broadcasted_iota(jnp.int32, (16,), 0)

    def fire(c):
        sl = pl.ds(c * CH, CH)
        pltpu.async_copy(utab_hbm.at[ublk_v.at[sl]], ubuf, sem)
        pltpu.async_copy(itab_hbm.at[iblk_v.at[sl]], ibuf, sem)

    def drain():
        pltpu.make_async_copy(utab_hbm.at[pl.ds(0, CH)], ubuf, sem).wait()
        pltpu.make_async_copy(itab_hbm.at[pl.ds(0, CH)], ibuf, sem).wait()

    def compute(c):
        cbase = c * CH

        @pl.loop(0, CH // 16)
        def _(g):
            uvec = uidx_v[pl.ds(cbase + g * 16, 16)]
            ivec = iidx_v[pl.ds(cbase + g * 16, 16)]
            acc = jnp.zeros((16,), jnp.float32)
            for j in range(16):
                b = g * 16 + j
                uo = (uvec[j] & 3) << 5
                io = (ivec[j] & 3) << 5
                u0 = ubuf[b, pl.ds(uo, 16)]
                u1 = ubuf[b, pl.ds(uo + 16, 16)]
                i0 = ibuf[b, pl.ds(io, 16)]
                i1 = ibuf[b, pl.ds(io + 16, 16)]
                r = jnp.sum(u0 * i0 + u1 * i1)
                acc = jnp.where(lane == j, r, acc)
            out_v[pl.ds(cbase + g * 16, 16)] = acc

    for c in range(NCHUNK):
        fire(c)
        drain()
        compute(c)

    pltpu.sync_copy(out_v, out_hbm.at[pl.ds(base, BPW)])


def kernel(user_ids, item_ids, user_table, item_table):
    return _sc_dot(user_ids.astype(jnp.int32), item_ids.astype(jnp.int32),
                   user_table.reshape(250000, 128),
                   item_table.reshape(250000, 128))


# speedup vs baseline: 1.5092x; 1.5092x over previous
import functools

import jax
import jax.numpy as jnp
from jax import lax
from jax.experimental import pallas as pl
from jax.experimental.pallas import tpu as pltpu
from jax.experimental.pallas import tpu_sc as plsc

B = 16384
D = 32
NC = 2
NS = 16
NW = NC * NS
BPW = B // NW
NCHUNK = 2
CH = BPW // NCHUNK
NSEM = 8
PER_SEM = 2 * CH // NSEM

_mesh = plsc.VectorSubcoreMesh(core_axis_name="c", subcore_axis_name="s")


@functools.partial(
    pl.kernel,
    mesh=_mesh,
    out_type=jax.ShapeDtypeStruct((B,), jnp.float32),
    scratch_types=[
        pltpu.VMEM((BPW,), jnp.int32),
        pltpu.VMEM((BPW,), jnp.int32),
        pltpu.VMEM((CH, D), jnp.float32),
        pltpu.VMEM((CH, D), jnp.float32),
        pltpu.VMEM((BPW,), jnp.float32),
        pltpu.SemaphoreType.DMA,
        [pltpu.SemaphoreType.DMA] * NSEM,
    ],
    compiler_params=pltpu.CompilerParams(needs_layout_passes=False),
)
def _sc_dot(uid_hbm, iid_hbm, utab_hbm, itab_hbm, out_hbm,
            uidx_v, iidx_v, ubuf, ibuf, out_v, sem_idx, sems):
    wid = lax.axis_index("s") * NC + lax.axis_index("c")
    base = wid * BPW
    pltpu.async_copy(uid_hbm.at[pl.ds(base, BPW)], uidx_v, sem_idx).wait()
    pltpu.async_copy(iid_hbm.at[pl.ds(base, BPW)], iidx_v, sem_idx).wait()

    lane = lax.broadcasted_iota(jnp.int32, (16,), 0)

    def fire(c):
        cbase = c * CH

        @pl.loop(0, CH // 16)
        def _(g):
            gb = cbase + g * 16
            uvec = uidx_v[pl.ds(gb, 16)]
            ivec = iidx_v[pl.ds(gb, 16)]
            for j in range(16):
                dst = g * 16 + j
                pltpu.async_copy(utab_hbm.at[pl.ds(uvec[j], 1)],
                                 ubuf.at[pl.ds(dst, 1)],
                                 sems[(2 * j) % NSEM])
                pltpu.async_copy(itab_hbm.at[pl.ds(ivec[j], 1)],
                                 ibuf.at[pl.ds(dst, 1)],
                                 sems[(2 * j + 1) % NSEM])

    def drain():
        for s in range(NSEM):
            pltpu.make_async_copy(utab_hbm.at[pl.ds(0, PER_SEM)],
                                  ubuf.at[pl.ds(0, PER_SEM)], sems[s]).wait()

    def compute(c):
        cbase = c * CH

        @pl.loop(0, CH // 16)
        def _(g):
            acc = jnp.zeros((16,), jnp.float32)
            for j in range(16):
                b = g * 16 + j
                u0 = ubuf[b, pl.ds(0, 16)]
                u1 = ubuf[b, pl.ds(16, 16)]
                i0 = ibuf[b, pl.ds(0, 16)]
                i1 = ibuf[b, pl.ds(16, 16)]
                r = jnp.sum(u0 * i0 + u1 * i1)
                acc = jnp.where(lane == j, r, acc)
            out_v[pl.ds(cbase + g * 16, 16)] = acc

    for c in range(NCHUNK):
        fire(c)
        drain()
        compute(c)

    pltpu.sync_copy(out_v, out_hbm.at[pl.ds(base, BPW)])


def kernel(user_ids, item_ids, user_table, item_table):
    return _sc_dot(user_ids.astype(jnp.int32), item_ids.astype(jnp.int32),
                   user_table, item_table)
